# Initial kernel scaffold; baseline (speedup 1.0000x reference)
import jax
import jax.numpy as jnp
from jax.experimental import pallas as pl


def kernel(subdomain_in, subdomain_out, x_idx, y_idx, z_idx, normal_x, normal_y, normal_z):
    raise NotImplementedError("write your pallas kernel here")


# speedup vs baseline: 2.4072x; 2.4072x over previous
import functools
import jax
import jax.numpy as jnp
from jax import lax
from jax.experimental import pallas as pl
from jax.experimental.pallas import tpu as pltpu
from jax.experimental.pallas import tpu_sc as plsc

_N = 128
_DX = 0.05
_WEIGHT = 10.0

_NC = 2
_NS = 16
_L = 16
_NW = _NC * _NS

_BATCH = 4
_GRID = _N * _N * _N
_OFFS = (0, -_N * _N, _N * _N, -_N, _N, -1, 1)
_NROW = len(_OFFS) * _BATCH


def _sc_body(n_valid, chunk, a_hbm, b_hbm, xi_hbm, yi_hbm, zi_hbm,
             nx_hbm, ny_hbm, nz_hbm, out_hbm,
             xi_v, yi_v, zi_v, nx_v, ny_v, nz_v,
             idx_v, va_v, vb_v, acc_v, sem, sem2):
    wid = lax.axis_index("s") * _NC + lax.axis_index("c")
    base = wid * chunk

    pltpu.sync_copy(xi_hbm.at[pl.ds(base, chunk)], xi_v)
    pltpu.sync_copy(yi_hbm.at[pl.ds(base, chunk)], yi_v)
    pltpu.sync_copy(zi_hbm.at[pl.ds(base, chunk)], zi_v)
    pltpu.sync_copy(nx_hbm.at[pl.ds(base, chunk)], nx_v)
    pltpu.sync_copy(ny_hbm.at[pl.ds(base, chunk)], ny_v)
    pltpu.sync_copy(nz_hbm.at[pl.ds(base, chunk)], nz_v)

    niter = chunk // _L

    def build(i, carry):
        s = i * _L
        x = xi_v[pl.ds(s, _L)]
        y = yi_v[pl.ds(s, _L)]
        z = zi_v[pl.ds(s, _L)]
        flat = x * (_N * _N) + y * _N + z
        for o, off in enumerate(_OFFS):
            for n in range(_BATCH):
                r = o * _BATCH + n
                idx_v[pl.ds(r * chunk + s, _L)] = flat + (n * _GRID + off)
        return carry

    lax.fori_loop(0, niter, build, 0)

    cp_a = pltpu.make_async_copy(a_hbm.at[idx_v], va_v, sem)
    cp_b = pltpu.make_async_copy(b_hbm.at[idx_v], vb_v, sem2)
    cp_a.start()
    cp_b.start()
    cp_a.wait()
    cp_b.wait()

    inv_dx = 1.0 / _DX

    def at(ref, o, n, s):
        return ref[pl.ds((o * _BATCH + n) * chunk + s, _L)]

    def compute(i, acc):
        s = i * _L
        glob = base + s + lax.iota(jnp.int32, _L)
        maskf = jnp.where(glob < n_valid, 1.0, 0.0).astype(jnp.float32)
        nx = nx_v[pl.ds(s, _L)]
        ny = ny_v[pl.ds(s, _L)]
        nz = nz_v[pl.ds(s, _L)]
        px = nx > 0.0
        py = ny > 0.0
        pz = nz > 0.0
        nzneg = nz < 0.0
        for n in range(_BATCH):
            c_in = at(va_v, 0, n, s)
            left_in = at(va_v, 1, n, s)
            right_in = at(va_v, 2, n, s)
            below_in = at(va_v, 3, n, s)
            above_in = at(va_v, 4, n, s)
            back_in = at(va_v, 5, n, s)
            front_in = at(va_v, 6, n, s)
            c_out = at(vb_v, 0, n, s)
            left_out = at(vb_v, 1, n, s)
            right_out = at(vb_v, 2, n, s)
            below_out = at(vb_v, 3, n, s)
            above_out = at(vb_v, 4, n, s)
            back_out = at(vb_v, 5, n, s)
            front_out = at(vb_v, 6, n, s)

            gx_in = jnp.where(px, c_in - left_in, right_in - c_in) * inv_dx
            gx_out = jnp.where(px, right_out - c_out, c_out - left_out) * inv_dx
            gy_in = jnp.where(py, c_in - below_in, above_in - c_in) * inv_dx
            gy_out = jnp.where(py, above_out - c_out, c_out - below_out) * inv_dx
            gz_in = jnp.where(pz, front_in - c_in, c_in - back_in) * inv_dx
            gz_out = jnp.where(nzneg, front_out - c_out, c_out - back_out) * inv_dx

            dc = c_in - c_out
            dnd = (gx_in - gx_out) * nx + (gy_in - gy_out) * ny \
                + (gz_in - gz_out) * nz
            acc = acc + maskf * (dc * dc + dnd * dnd)
        return acc

    acc = lax.fori_loop(0, niter, compute,
                        jnp.zeros((_L,), jnp.float32))
    acc_v[...] = acc
    pltpu.sync_copy(acc_v, out_hbm.at[wid])


def kernel(subdomain_in, subdomain_out, x_idx, y_idx, z_idx,
           normal_x, normal_y, normal_z):
    k = x_idx.shape[0]
    chunk = ((k + _NW - 1) // _NW + _L - 1) // _L * _L
    kp = chunk * _NW
    pad = kp - k

    a = subdomain_in[:, 0].reshape(-1)
    b = subdomain_out[:, 0].reshape(-1)
    xi = jnp.pad(x_idx, (0, pad), constant_values=64)
    yi = jnp.pad(y_idx, (0, pad), constant_values=64)
    zi = jnp.pad(z_idx, (0, pad), constant_values=64)
    nx = jnp.pad(normal_x, (0, pad))
    ny = jnp.pad(normal_y, (0, pad))
    nz = jnp.pad(normal_z, (0, pad))

    mesh = plsc.VectorSubcoreMesh(core_axis_name="c", subcore_axis_name="s")
    fn = pl.kernel(
        functools.partial(_sc_body, k, chunk),
        out_type=jax.ShapeDtypeStruct((_NW, _L), jnp.float32),
        mesh=mesh,
        scratch_types=[
            pltpu.VMEM((chunk,), jnp.int32),
            pltpu.VMEM((chunk,), jnp.int32),
            pltpu.VMEM((chunk,), jnp.int32),
            pltpu.VMEM((chunk,), jnp.float32),
            pltpu.VMEM((chunk,), jnp.float32),
            pltpu.VMEM((chunk,), jnp.float32),
            pltpu.VMEM((_NROW * chunk,), jnp.int32),
            pltpu.VMEM((_NROW * chunk,), jnp.float32),
            pltpu.VMEM((_NROW * chunk,), jnp.float32),
            pltpu.VMEM((_L,), jnp.float32),
            pltpu.SemaphoreType.DMA,
            pltpu.SemaphoreType.DMA,
        ],
    )
    partial = fn(a, b, xi, yi, zi, nx, ny, nz)
    scale = _WEIGHT / (_BATCH * k)
    return jnp.sum(partial) * scale


# speedup vs baseline: 2.7882x; 1.1583x over previous
import functools
import jax
import jax.numpy as jnp
from jax import lax
from jax.experimental import pallas as pl
from jax.experimental.pallas import tpu as pltpu
from jax.experimental.pallas import tpu_sc as plsc

_N = 128
_DX = 0.05
_WEIGHT = 10.0

_NC = 2
_NS = 16
_L = 16
_NW = _NC * _NS

_BATCH = 4
_GRID = _N * _N * _N
_OFFS = (0, -_N * _N, _N * _N, -_N, _N, -1, 1)
_NROW = len(_OFFS) * _BATCH
_NSUB = 4


def _sc_body(n_valid, chunk, a_hbm, b_hbm, side_hbm, nrm_hbm, out_hbm,
             side_v, nrm_v, idx_v, va_v, vb_v, acc_v, *sems):
    wid = lax.axis_index("s") * _NC + lax.axis_index("c")
    base = wid * chunk
    sub = chunk // _NSUB

    pltpu.sync_copy(side_hbm.at[:, pl.ds(base, chunk)], side_v)
    pltpu.sync_copy(nrm_hbm.at[:, pl.ds(base, chunk)], nrm_v)

    inv_dx = 1.0 / _DX
    copies = []
    for j in range(_NSUB):
        def build(ii, carry, j=j):
            s = j * sub + ii * _L
            x = side_v[0, pl.ds(s, _L)]
            y = side_v[1, pl.ds(s, _L)]
            z = side_v[2, pl.ds(s, _L)]
            flat = x * (_N * _N) + y * _N + z
            for o, off in enumerate(_OFFS):
                for n in range(_BATCH):
                    r = o * _BATCH + n
                    idx_v[pl.ds((j * _NROW + r) * sub + ii * _L, _L)] = \
                        flat + (n * _GRID + off)
            return carry

        lax.fori_loop(0, sub // _L, build, 0)
        sl = pl.ds(j * _NROW * sub, _NROW * sub)
        cp_a = pltpu.make_async_copy(a_hbm.at[idx_v.at[sl]], va_v.at[sl],
                                     sems[2 * j])
        cp_b = pltpu.make_async_copy(b_hbm.at[idx_v.at[sl]], vb_v.at[sl],
                                     sems[2 * j + 1])
        cp_a.start()
        cp_b.start()
        copies.append((cp_a, cp_b))

    def at(ref, j, o, n, s):
        return ref[pl.ds((j * _NROW + o * _BATCH + n) * sub + s, _L)]

    acc = jnp.zeros((_L,), jnp.float32)
    for j in range(_NSUB):
        cp_a, cp_b = copies[j]
        cp_a.wait()
        cp_b.wait()

        def compute(ii, acc, j=j):
            s = ii * _L
            glob = base + j * sub + s + lax.iota(jnp.int32, _L)
            maskf = jnp.where(glob < n_valid, 1.0, 0.0).astype(jnp.float32)
            nx = nrm_v[0, pl.ds(j * sub + s, _L)]
            ny = nrm_v[1, pl.ds(j * sub + s, _L)]
            nz = nrm_v[2, pl.ds(j * sub + s, _L)]
            px = nx > 0.0
            py = ny > 0.0
            pz = nz > 0.0
            nzneg = nz < 0.0
            for n in range(_BATCH):
                c_in = at(va_v, j, 0, n, s)
                left_in = at(va_v, j, 1, n, s)
                right_in = at(va_v, j, 2, n, s)
                below_in = at(va_v, j, 3, n, s)
                above_in = at(va_v, j, 4, n, s)
                back_in = at(va_v, j, 5, n, s)
                front_in = at(va_v, j, 6, n, s)
                c_out = at(vb_v, j, 0, n, s)
                left_out = at(vb_v, j, 1, n, s)
                right_out = at(vb_v, j, 2, n, s)
                below_out = at(vb_v, j, 3, n, s)
                above_out = at(vb_v, j, 4, n, s)
                back_out = at(vb_v, j, 5, n, s)
                front_out = at(vb_v, j, 6, n, s)

                gx_in = jnp.where(px, c_in - left_in, right_in - c_in)
                gx_out = jnp.where(px, right_out - c_out, c_out - left_out)
                gy_in = jnp.where(py, c_in - below_in, above_in - c_in)
                gy_out = jnp.where(py, above_out - c_out, c_out - below_out)
                gz_in = jnp.where(pz, front_in - c_in, c_in - back_in)
                gz_out = jnp.where(nzneg, front_out - c_out, c_out - back_out)

                dc = c_in - c_out
                dnd = ((gx_in - gx_out) * nx + (gy_in - gy_out) * ny
                       + (gz_in - gz_out) * nz) * inv_dx
                acc = acc + maskf * (dc * dc + dnd * dnd)
            return acc

        acc = lax.fori_loop(0, sub // _L, compute, acc)

    acc_v[...] = acc
    pltpu.sync_copy(acc_v, out_hbm.at[wid])


def kernel(subdomain_in, subdomain_out, x_idx, y_idx, z_idx,
           normal_x, normal_y, normal_z):
    k = x_idx.shape[0]
    q = _L * _NSUB
    chunk = ((k + _NW - 1) // _NW + q - 1) // q * q
    kp = chunk * _NW
    pad = kp - k

    a = subdomain_in[:, 0].reshape(-1)
    b = subdomain_out[:, 0].reshape(-1)
    side = jnp.stack([x_idx, y_idx, z_idx])
    side = jnp.pad(side, ((0, 0), (0, pad)), constant_values=64)
    nrm = jnp.pad(jnp.stack([normal_x, normal_y, normal_z]),
                  ((0, 0), (0, pad)))

    mesh = plsc.VectorSubcoreMesh(core_axis_name="c", subcore_axis_name="s")
    fn = pl.kernel(
        functools.partial(_sc_body, k, chunk),
        out_type=jax.ShapeDtypeStruct((_NW, _L), jnp.float32),
        mesh=mesh,
        scratch_types=[
            pltpu.VMEM((3, chunk), jnp.int32),
            pltpu.VMEM((3, chunk), jnp.float32),
            pltpu.VMEM((_NROW * chunk,), jnp.int32),
            pltpu.VMEM((_NROW * chunk,), jnp.float32),
            pltpu.VMEM((_NROW * chunk,), jnp.float32),
            pltpu.VMEM((_L,), jnp.float32),
        ] + [pltpu.SemaphoreType.DMA] * (2 * _NSUB),
    )
    partial = fn(a, b, side, nrm)
    scale = _WEIGHT / (_BATCH * k)
    return jnp.sum(partial) * scale


# speedup vs baseline: 2.8325x; 1.0159x over previous
import functools
import jax
import jax.numpy as jnp
from jax import lax
from jax.experimental import pallas as pl
from jax.experimental.pallas import tpu as pltpu
from jax.experimental.pallas import tpu_sc as plsc

_N = 128
_DX = 0.05
_WEIGHT = 10.0

_NC = 2
_NS = 16
_L = 16
_NW = _NC * _NS

_BATCH = 4
_GRID = _N * _N * _N
_OFFS = (0, -_N * _N, _N * _N, -_N, _N, -1, 1)
_NROW = len(_OFFS) * _BATCH
_NSUB = 8


def _sc_body(n_valid, chunk, a_hbm, b_hbm, side_hbm, nrm_hbm, out_hbm,
             side_v, nrm_v, idx_v, va_v, vb_v, acc_v, *sems):
    wid = lax.axis_index("s") * _NC + lax.axis_index("c")
    base = wid * chunk
    sub = chunk // _NSUB

    pltpu.sync_copy(side_hbm.at[:, pl.ds(base, chunk)], side_v)
    pltpu.sync_copy(nrm_hbm.at[:, pl.ds(base, chunk)], nrm_v)

    inv_dx = 1.0 / _DX
    copies = []
    for j in range(_NSUB):
        def build(ii, carry, j=j):
            s = j * sub + ii * _L
            x = side_v[0, pl.ds(s, _L)]
            y = side_v[1, pl.ds(s, _L)]
            z = side_v[2, pl.ds(s, _L)]
            flat = x * (_N * _N) + y * _N + z
            for o, off in enumerate(_OFFS):
                for n in range(_BATCH):
                    r = o * _BATCH + n
                    idx_v[pl.ds((j * _NROW + r) * sub + ii * _L, _L)] = \
                        flat + (n * _GRID + off)
            return carry

        lax.fori_loop(0, sub // _L, build, 0)
        sl = pl.ds(j * _NROW * sub, _NROW * sub)
        cp_a = pltpu.make_async_copy(a_hbm.at[idx_v.at[sl]], va_v.at[sl],
                                     sems[2 * j])
        cp_b = pltpu.make_async_copy(b_hbm.at[idx_v.at[sl]], vb_v.at[sl],
                                     sems[2 * j + 1])
        cp_a.start()
        cp_b.start()
        copies.append((cp_a, cp_b))

    def at(ref, j, o, n, s):
        return ref[pl.ds((j * _NROW + o * _BATCH + n) * sub + s, _L)]

    acc = jnp.zeros((_L,), jnp.float32)
    for j in range(_NSUB):
        cp_a, cp_b = copies[j]
        cp_a.wait()
        cp_b.wait()

        def compute(ii, acc, j=j):
            s = ii * _L
            glob = base + j * sub + s + lax.iota(jnp.int32, _L)
            maskf = jnp.where(glob < n_valid, 1.0, 0.0).astype(jnp.float32)
            nx = nrm_v[0, pl.ds(j * sub + s, _L)]
            ny = nrm_v[1, pl.ds(j * sub + s, _L)]
            nz = nrm_v[2, pl.ds(j * sub + s, _L)]
            px = nx > 0.0
            py = ny > 0.0
            pz = nz > 0.0
            nzneg = nz < 0.0
            for n in range(_BATCH):
                c_in = at(va_v, j, 0, n, s)
                left_in = at(va_v, j, 1, n, s)
                right_in = at(va_v, j, 2, n, s)
                below_in = at(va_v, j, 3, n, s)
                above_in = at(va_v, j, 4, n, s)
                back_in = at(va_v, j, 5, n, s)
                front_in = at(va_v, j, 6, n, s)
                c_out = at(vb_v, j, 0, n, s)
                left_out = at(vb_v, j, 1, n, s)
                right_out = at(vb_v, j, 2, n, s)
                below_out = at(vb_v, j, 3, n, s)
                above_out = at(vb_v, j, 4, n, s)
                back_out = at(vb_v, j, 5, n, s)
                front_out = at(vb_v, j, 6, n, s)

                gx_in = jnp.where(px, c_in - left_in, right_in - c_in)
                gx_out = jnp.where(px, right_out - c_out, c_out - left_out)
                gy_in = jnp.where(py, c_in - below_in, above_in - c_in)
                gy_out = jnp.where(py, above_out - c_out, c_out - below_out)
                gz_in = jnp.where(pz, front_in - c_in, c_in - back_in)
                gz_out = jnp.where(nzneg, front_out - c_out, c_out - back_out)

                dc = c_in - c_out
                dnd = ((gx_in - gx_out) * nx + (gy_in - gy_out) * ny
                       + (gz_in - gz_out) * nz) * inv_dx
                acc = acc + maskf * (dc * dc + dnd * dnd)
            return acc

        acc = lax.fori_loop(0, sub // _L, compute, acc)

    acc_v[...] = acc
    pltpu.sync_copy(acc_v, out_hbm.at[wid])


def kernel(subdomain_in, subdomain_out, x_idx, y_idx, z_idx,
           normal_x, normal_y, normal_z):
    k = x_idx.shape[0]
    q = _L * _NSUB
    chunk = ((k + _NW - 1) // _NW + q - 1) // q * q
    kp = chunk * _NW
    pad = kp - k

    a = subdomain_in[:, 0].reshape(-1)
    b = subdomain_out[:, 0].reshape(-1)
    side = jnp.stack([x_idx, y_idx, z_idx])
    side = jnp.pad(side, ((0, 0), (0, pad)), constant_values=64)
    nrm = jnp.pad(jnp.stack([normal_x, normal_y, normal_z]),
                  ((0, 0), (0, pad)))

    mesh = plsc.VectorSubcoreMesh(core_axis_name="c", subcore_axis_name="s")
    fn = pl.kernel(
        functools.partial(_sc_body, k, chunk),
        out_type=jax.ShapeDtypeStruct((_NW, _L), jnp.float32),
        mesh=mesh,
        scratch_types=[
            pltpu.VMEM((3, chunk), jnp.int32),
            pltpu.VMEM((3, chunk), jnp.float32),
            pltpu.VMEM((_NROW * chunk,), jnp.int32),
            pltpu.VMEM((_NROW * chunk,), jnp.float32),
            pltpu.VMEM((_NROW * chunk,), jnp.float32),
            pltpu.VMEM((_L,), jnp.float32),
        ] + [pltpu.SemaphoreType.DMA] * (2 * _NSUB),
    )
    partial = fn(a, b, side, nrm)
    scale = _WEIGHT / (_BATCH * k)
    return jnp.sum(partial) * scale


# speedup vs baseline: 2.9409x; 1.0383x over previous
import functools
import jax
import jax.numpy as jnp
from jax import lax
from jax.experimental import pallas as pl
from jax.experimental.pallas import tpu as pltpu
from jax.experimental.pallas import tpu_sc as plsc

_N = 128
_DX = 0.05
_WEIGHT = 10.0

_NC = 2
_NS = 16
_L = 16
_NW = _NC * _NS

_BATCH = 4
_GRID = _N * _N * _N
_OFFS = (0, -_N * _N, _N * _N, -_N, _N, -1, 1)
_NROW = len(_OFFS) * _BATCH
_NSUB = 8


def _sc_body(n_valid, chunk, a_hbm, b_hbm, side_hbm, nrm_hbm, out_hbm,
             side_v, nrm_v, idx_v, va_v, vb_v, acc_v, *sems):
    wid = lax.axis_index("s") * _NC + lax.axis_index("c")
    base = wid * chunk
    sub = chunk // _NSUB

    pltpu.sync_copy(side_hbm.at[:, pl.ds(base, chunk)], side_v)
    pltpu.sync_copy(nrm_hbm.at[:, pl.ds(base, chunk)], nrm_v)

    inv_dx = 1.0 / _DX
    copies = []
    for j in range(_NSUB):
        def build(ii, carry, j=j):
            s = j * sub + ii * _L
            flat = side_v[0, pl.ds(s, _L)]
            for o, off in enumerate(_OFFS):
                for n in range(_BATCH):
                    r = o * _BATCH + n
                    idx_v[pl.ds((j * _NROW + r) * sub + ii * _L, _L)] = \
                        flat + (n * _GRID + off)
            return carry

        lax.fori_loop(0, sub // _L, build, 0)
        sl = pl.ds(j * _NROW * sub, _NROW * sub)
        cp_a = pltpu.make_async_copy(a_hbm.at[idx_v.at[sl]], va_v.at[sl],
                                     sems[2 * j])
        cp_b = pltpu.make_async_copy(b_hbm.at[idx_v.at[sl]], vb_v.at[sl],
                                     sems[2 * j + 1])
        cp_a.start()
        cp_b.start()
        copies.append((cp_a, cp_b))

    def at(ref, j, o, n, s):
        return ref[pl.ds((j * _NROW + o * _BATCH + n) * sub + s, _L)]

    acc = jnp.zeros((_L,), jnp.float32)
    for j in range(_NSUB):
        cp_a, cp_b = copies[j]
        cp_a.wait()
        cp_b.wait()

        def compute(ii, acc, j=j):
            s = ii * _L
            glob = base + j * sub + s + lax.iota(jnp.int32, _L)
            maskf = jnp.where(glob < n_valid, 1.0, 0.0).astype(jnp.float32)
            nx = nrm_v[0, pl.ds(j * sub + s, _L)]
            ny = nrm_v[1, pl.ds(j * sub + s, _L)]
            nz = nrm_v[2, pl.ds(j * sub + s, _L)]
            px = nx > 0.0
            py = ny > 0.0
            pz = nz > 0.0
            nzneg = nz < 0.0
            for n in range(_BATCH):
                c_in = at(va_v, j, 0, n, s)
                left_in = at(va_v, j, 1, n, s)
                right_in = at(va_v, j, 2, n, s)
                below_in = at(va_v, j, 3, n, s)
                above_in = at(va_v, j, 4, n, s)
                back_in = at(va_v, j, 5, n, s)
                front_in = at(va_v, j, 6, n, s)
                c_out = at(vb_v, j, 0, n, s)
                left_out = at(vb_v, j, 1, n, s)
                right_out = at(vb_v, j, 2, n, s)
                below_out = at(vb_v, j, 3, n, s)
                above_out = at(vb_v, j, 4, n, s)
                back_out = at(vb_v, j, 5, n, s)
                front_out = at(vb_v, j, 6, n, s)

                gx_in = jnp.where(px, c_in - left_in, right_in - c_in)
                gx_out = jnp.where(px, right_out - c_out, c_out - left_out)
                gy_in = jnp.where(py, c_in - below_in, above_in - c_in)
                gy_out = jnp.where(py, above_out - c_out, c_out - below_out)
                gz_in = jnp.where(pz, front_in - c_in, c_in - back_in)
                gz_out = jnp.where(nzneg, front_out - c_out, c_out - back_out)

                dc = c_in - c_out
                dnd = ((gx_in - gx_out) * nx + (gy_in - gy_out) * ny
                       + (gz_in - gz_out) * nz) * inv_dx
                acc = acc + maskf * (dc * dc + dnd * dnd)
            return acc

        acc = lax.fori_loop(0, sub // _L, compute, acc)

    acc_v[...] = acc
    pltpu.sync_copy(acc_v, out_hbm.at[wid])


def kernel(subdomain_in, subdomain_out, x_idx, y_idx, z_idx,
           normal_x, normal_y, normal_z):
    k = x_idx.shape[0]
    q = _L * _NSUB
    chunk = ((k + _NW - 1) // _NW + q - 1) // q * q
    kp = chunk * _NW
    pad = kp - k

    a = subdomain_in[:, 0].reshape(-1)
    b = subdomain_out[:, 0].reshape(-1)
    flat = x_idx * (_N * _N) + y_idx * _N + z_idx
    side = jnp.pad(flat[None, :], ((0, 0), (0, pad)),
                   constant_values=64 * (_N * _N) + 64 * _N + 64)
    nrm = jnp.pad(jnp.stack([normal_x, normal_y, normal_z]),
                  ((0, 0), (0, pad)))

    mesh = plsc.VectorSubcoreMesh(core_axis_name="c", subcore_axis_name="s")
    fn = pl.kernel(
        functools.partial(_sc_body, k, chunk),
        out_type=jax.ShapeDtypeStruct((_NW, _L), jnp.float32),
        mesh=mesh,
        scratch_types=[
            pltpu.VMEM((1, chunk), jnp.int32),
            pltpu.VMEM((3, chunk), jnp.float32),
            pltpu.VMEM((_NROW * chunk,), jnp.int32),
            pltpu.VMEM((_NROW * chunk,), jnp.float32),
            pltpu.VMEM((_NROW * chunk,), jnp.float32),
            pltpu.VMEM((_L,), jnp.float32),
        ] + [pltpu.SemaphoreType.DMA] * (2 * _NSUB),
    )
    partial = fn(a, b, side, nrm)
    scale = _WEIGHT / (_BATCH * k)
    return jnp.sum(partial) * scale


# speedup vs baseline: 2.9602x; 1.0066x over previous
import functools
import jax
import jax.numpy as jnp
from jax import lax
from jax.experimental import pallas as pl
from jax.experimental.pallas import tpu as pltpu
from jax.experimental.pallas import tpu_sc as plsc

_N = 128
_DX = 0.05
_WEIGHT = 10.0

_NC = 2
_NS = 16
_L = 16
_NW = _NC * _NS

_BATCH = 4
_GRID = _N * _N * _N
_OFFS = (0, -_N * _N, _N * _N, -_N, _N, -1, 1)
_NROW = len(_OFFS) * _BATCH
_NSUB = 8


def _sc_body(n_valid, chunk, a_hbm, b_hbm, side_hbm, nrm_hbm, out_hbm,
             side_v, nrm_v, idx_v, va_v, vb_v, acc_v, *sems):
    wid = lax.axis_index("s") * _NC + lax.axis_index("c")
    base = wid * chunk
    sub = chunk // _NSUB

    nrm_cp = pltpu.make_async_copy(nrm_hbm.at[:, pl.ds(base, chunk)],
                                   nrm_v, sems[2 * _NSUB])
    nrm_cp.start()
    pltpu.sync_copy(side_hbm.at[pl.ds(base, chunk)], side_v)

    inv_dx = 1.0 / _DX
    copies = []
    for j in range(_NSUB):
        def build(ii, carry, j=j):
            s = j * sub + ii * _L
            flat = side_v[pl.ds(s, _L)]
            for o, off in enumerate(_OFFS):
                for n in range(_BATCH):
                    r = o * _BATCH + n
                    idx_v[pl.ds((j * _NROW + r) * sub + ii * _L, _L)] = \
                        flat + (n * _GRID + off)
            return carry

        lax.fori_loop(0, sub // _L, build, 0)
        sl = pl.ds(j * _NROW * sub, _NROW * sub)
        cp_a = pltpu.make_async_copy(a_hbm.at[idx_v.at[sl]], va_v.at[sl],
                                     sems[2 * j])
        cp_b = pltpu.make_async_copy(b_hbm.at[idx_v.at[sl]], vb_v.at[sl],
                                     sems[2 * j + 1])
        cp_a.start()
        cp_b.start()
        copies.append((cp_a, cp_b))

    def at(ref, j, o, n, s):
        return ref[pl.ds((j * _NROW + o * _BATCH + n) * sub + s, _L)]

    nrm_cp.wait()
    acc = jnp.zeros((_L,), jnp.float32)
    for j in range(_NSUB):
        cp_a, cp_b = copies[j]
        cp_a.wait()
        cp_b.wait()

        def compute(ii, acc, j=j):
            s = ii * _L
            glob = base + j * sub + s + lax.iota(jnp.int32, _L)
            maskf = jnp.where(glob < n_valid, 1.0, 0.0).astype(jnp.float32)
            nx = nrm_v[0, pl.ds(j * sub + s, _L)]
            ny = nrm_v[1, pl.ds(j * sub + s, _L)]
            nz = nrm_v[2, pl.ds(j * sub + s, _L)]
            px = nx > 0.0
            py = ny > 0.0
            pz = nz > 0.0
            nzneg = nz < 0.0
            for n in range(_BATCH):
                c_in = at(va_v, j, 0, n, s)
                left_in = at(va_v, j, 1, n, s)
                right_in = at(va_v, j, 2, n, s)
                below_in = at(va_v, j, 3, n, s)
                above_in = at(va_v, j, 4, n, s)
                back_in = at(va_v, j, 5, n, s)
                front_in = at(va_v, j, 6, n, s)
                c_out = at(vb_v, j, 0, n, s)
                left_out = at(vb_v, j, 1, n, s)
                right_out = at(vb_v, j, 2, n, s)
                below_out = at(vb_v, j, 3, n, s)
                above_out = at(vb_v, j, 4, n, s)
                back_out = at(vb_v, j, 5, n, s)
                front_out = at(vb_v, j, 6, n, s)

                gx_in = jnp.where(px, c_in - left_in, right_in - c_in)
                gx_out = jnp.where(px, right_out - c_out, c_out - left_out)
                gy_in = jnp.where(py, c_in - below_in, above_in - c_in)
                gy_out = jnp.where(py, above_out - c_out, c_out - below_out)
                gz_in = jnp.where(pz, front_in - c_in, c_in - back_in)
                gz_out = jnp.where(nzneg, front_out - c_out, c_out - back_out)

                dc = c_in - c_out
                dnd = ((gx_in - gx_out) * nx + (gy_in - gy_out) * ny
                       + (gz_in - gz_out) * nz) * inv_dx
                acc = acc + maskf * (dc * dc + dnd * dnd)
            return acc

        acc = lax.fori_loop(0, sub // _L, compute, acc)

    acc_v[...] = acc
    pltpu.sync_copy(acc_v, out_hbm.at[wid])


def kernel(subdomain_in, subdomain_out, x_idx, y_idx, z_idx,
           normal_x, normal_y, normal_z):
    k = x_idx.shape[0]
    q = _L * _NSUB
    chunk = ((k + _NW - 1) // _NW + q - 1) // q * q
    kp = chunk * _NW
    pad = kp - k

    a = subdomain_in[:, 0].reshape(-1)
    b = subdomain_out[:, 0].reshape(-1)
    flat = x_idx * (_N * _N) + y_idx * _N + z_idx
    side = jnp.pad(flat, (0, pad),
                   constant_values=64 * (_N * _N) + 64 * _N + 64)
    nrm = jnp.pad(jnp.stack([normal_x, normal_y, normal_z]),
                  ((0, 0), (0, pad)))

    mesh = plsc.VectorSubcoreMesh(core_axis_name="c", subcore_axis_name="s")
    fn = pl.kernel(
        functools.partial(_sc_body, k, chunk),
        out_type=jax.ShapeDtypeStruct((_NW, _L), jnp.float32),
        mesh=mesh,
        scratch_types=[
            pltpu.VMEM((chunk,), jnp.int32),
            pltpu.VMEM((3, chunk), jnp.float32),
            pltpu.VMEM((_NROW * chunk,), jnp.int32),
            pltpu.VMEM((_NROW * chunk,), jnp.float32),
            pltpu.VMEM((_NROW * chunk,), jnp.float32),
            pltpu.VMEM((_L,), jnp.float32),
        ] + [pltpu.SemaphoreType.DMA] * (2 * _NSUB + 1),
    )
    partial = fn(a, b, side, nrm)
    scale = _WEIGHT / (_BATCH * k)
    return jnp.sum(partial) * scale
